# baseline (speedup 1.0000x reference)
import functools

import jax
import jax.numpy as jnp
from jax import lax
from jax.experimental import pallas as pl
from jax.experimental.pallas import tpu as pltpu
from jax.experimental.pallas import tpu_sc as plsc

LATENT_DIM = 64
LANES = 16
NBUF = 4


@jax.jit
def _mf_forward(u_start, u_col, i_start, i_col, users_t, items_t):
    batch = u_start.shape[0]
    info = plsc.get_sparse_core_info()
    nw = info.num_cores * info.num_subcores
    bpw = batch // nw
    n_groups = bpw // LANES
    mesh = plsc.VectorSubcoreMesh(core_axis_name="c", subcore_axis_name="s")

    @functools.partial(
        pl.kernel,
        out_type=jax.ShapeDtypeStruct((batch,), jnp.float32),
        mesh=mesh,
        compiler_params=pltpu.CompilerParams(needs_layout_passes=False),
        scratch_types=[
            pltpu.VMEM((bpw + LANES,), jnp.int32),
            pltpu.VMEM((bpw + LANES,), jnp.int32),
            pltpu.VMEM((bpw + LANES,), jnp.int32),
            pltpu.VMEM((bpw + LANES,), jnp.int32),
            pltpu.VMEM((NBUF, LATENT_DIM, 128), jnp.float32),
            pltpu.VMEM((NBUF, LATENT_DIM, 128), jnp.float32),
            pltpu.VMEM((LANES, LANES), jnp.float32),
            pltpu.VMEM((bpw,), jnp.float32),
        ] + [pltpu.SemaphoreType.DMA] * NBUF,
    )
    def kern(us_hbm, uc_hbm, is_hbm, ic_hbm, users_hbm, items_hbm, out_hbm,
             us_v, uc_v, is_v, ic_v, ubufs, ibufs, stage, out_v, *sems):
        wid = lax.axis_index("s") * info.num_cores + lax.axis_index("c")
        base = wid * bpw

        pltpu.sync_copy(us_hbm.at[pl.ds(base, bpw)], us_v.at[pl.ds(0, bpw)])
        pltpu.sync_copy(uc_hbm.at[pl.ds(base, bpw)], uc_v.at[pl.ds(0, bpw)])
        pltpu.sync_copy(is_hbm.at[pl.ds(base, bpw)], is_v.at[pl.ds(0, bpw)])
        pltpu.sync_copy(ic_hbm.at[pl.ds(base, bpw)], ic_v.at[pl.ds(0, bpw)])

        lanes_iota = lax.iota(jnp.int32, LANES)

        def fire(slot, k):
            us = us_v[pl.ds(k, LANES)][0]
            its = is_v[pl.ds(k, LANES)][0]
            pltpu.async_copy(
                users_hbm.at[:, pl.ds(pl.multiple_of(us, 128), 128)],
                ubufs.at[slot], sems[slot])
            pltpu.async_copy(
                items_hbm.at[:, pl.ds(pl.multiple_of(its, 128), 128)],
                ibufs.at[slot], sems[slot])

        def wait(slot):
            for _ in range(2):
                pltpu.make_async_copy(
                    users_hbm.at[:, pl.ds(0, 128)], ubufs.at[slot],
                    sems[slot]).wait()

        for slot in range(NBUF):
            fire(slot, slot)

        def group_body(g, _):
            k0 = g * LANES
            for t in range(LANES):
                slot = t % NBUF
                k = k0 + t
                wait(slot)
                uc = uc_v[pl.ds(k, LANES)][0]
                ic = ic_v[pl.ds(k, LANES)][0]
                ucvec = lanes_iota * 0 + uc
                icvec = lanes_iota * 0 + ic
                acc = None
                for q in range(LATENT_DIM // LANES):
                    rows = q * LANES + lanes_iota
                    pu = plsc.load_gather(ubufs.at[slot], [rows, ucvec])
                    pi = plsc.load_gather(ibufs.at[slot], [rows, icvec])
                    prod = pu * pi
                    acc = prod if acc is None else acc + prod
                stage[t, :] = acc
                fire(slot, jnp.minimum(k + NBUF, bpw - 1))
            tot = None
            for c in range(LANES):
                cvec = lanes_iota * 0 + c
                col = plsc.load_gather(stage, [lanes_iota, cvec])
                tot = col if tot is None else tot + col
            out_v[pl.ds(k0, LANES)] = tot
            return 0

        lax.fori_loop(0, n_groups, group_body, 0)

        for slot in range(NBUF):
            wait(slot)

        pltpu.sync_copy(out_v, out_hbm.at[pl.ds(base, bpw)])

    return kern(u_start, u_col, i_start, i_col, users_t, items_t)


def kernel(x, users_weight, items_weight):
    x32 = x.astype(jnp.int32)
    u = x32[:, 0]
    it = x32[:, 1]
    return _mf_forward(u & ~127, u & 127, it & ~127, it & 127,
                       users_weight.T, items_weight.T)


# speedup vs baseline: 1.0021x; 1.0021x over previous
import functools

import jax
import jax.numpy as jnp
from jax import lax
from jax.experimental import pallas as pl
from jax.experimental.pallas import tpu as pltpu
from jax.experimental.pallas import tpu_sc as plsc

LATENT_DIM = 64
LANES = 16
NBUF = 4


@jax.jit
def _mf_forward(u_start, u_col, i_start, i_col, users_t, items_t):
    batch = u_start.shape[0]
    info = plsc.get_sparse_core_info()
    nw = info.num_cores * info.num_subcores
    bpw = batch // nw
    n_groups = bpw // LANES
    mesh = plsc.VectorSubcoreMesh(core_axis_name="c", subcore_axis_name="s")

    @functools.partial(
        pl.kernel,
        out_type=jax.ShapeDtypeStruct((batch,), jnp.float32),
        mesh=mesh,
        compiler_params=pltpu.CompilerParams(needs_layout_passes=False),
        scratch_types=[
            pltpu.VMEM((bpw + LANES,), jnp.int32),
            pltpu.VMEM((bpw + LANES,), jnp.int32),
            pltpu.VMEM((bpw + LANES,), jnp.int32),
            pltpu.VMEM((bpw + LANES,), jnp.int32),
            pltpu.VMEM((NBUF, LATENT_DIM, 128), jnp.float32),
            pltpu.VMEM((NBUF, LATENT_DIM, 128), jnp.float32),
            pltpu.VMEM((LANES, LANES), jnp.float32),
            pltpu.VMEM((bpw,), jnp.float32),
        ] + [pltpu.SemaphoreType.DMA] * NBUF,
    )
    def kern(us_hbm, uc_hbm, is_hbm, ic_hbm, users_hbm, items_hbm, out_hbm,
             us_v, uc_v, is_v, ic_v, ubufs, ibufs, stage, out_v, *sems):
        wid = lax.axis_index("s") * info.num_cores + lax.axis_index("c")
        base = wid * bpw

        pltpu.sync_copy(us_hbm.at[pl.ds(base, bpw)], us_v.at[pl.ds(0, bpw)])
        pltpu.sync_copy(uc_hbm.at[pl.ds(base, bpw)], uc_v.at[pl.ds(0, bpw)])
        pltpu.sync_copy(is_hbm.at[pl.ds(base, bpw)], is_v.at[pl.ds(0, bpw)])
        pltpu.sync_copy(ic_hbm.at[pl.ds(base, bpw)], ic_v.at[pl.ds(0, bpw)])

        lanes_iota = lax.iota(jnp.int32, LANES)

        HALF = LATENT_DIM // 2

        def fire(slot, k):
            us = us_v[pl.ds(k, LANES)][0]
            its = is_v[pl.ds(k, LANES)][0]
            for h in range(2):
                pltpu.async_copy(
                    users_hbm.at[pl.ds(h * HALF, HALF),
                                 pl.ds(pl.multiple_of(us, 128), 128)],
                    ubufs.at[slot, pl.ds(h * HALF, HALF)], sems[slot])
                pltpu.async_copy(
                    items_hbm.at[pl.ds(h * HALF, HALF),
                                 pl.ds(pl.multiple_of(its, 128), 128)],
                    ibufs.at[slot, pl.ds(h * HALF, HALF)], sems[slot])

        def wait(slot):
            for _ in range(4):
                pltpu.make_async_copy(
                    users_hbm.at[pl.ds(0, HALF), pl.ds(0, 128)],
                    ubufs.at[slot, pl.ds(0, HALF)], sems[slot]).wait()

        for slot in range(NBUF):
            fire(slot, slot)

        def group_body(g, _):
            k0 = g * LANES
            for t in range(LANES):
                slot = t % NBUF
                k = k0 + t
                wait(slot)
                uc = uc_v[pl.ds(k, LANES)][0]
                ic = ic_v[pl.ds(k, LANES)][0]
                ucvec = lanes_iota * 0 + uc
                icvec = lanes_iota * 0 + ic
                acc = None
                for q in range(LATENT_DIM // LANES):
                    rows = q * LANES + lanes_iota
                    pu = plsc.load_gather(ubufs.at[slot], [rows, ucvec])
                    pi = plsc.load_gather(ibufs.at[slot], [rows, icvec])
                    prod = pu * pi
                    acc = prod if acc is None else acc + prod
                stage[t, :] = acc
                fire(slot, jnp.minimum(k + NBUF, bpw - 1))
            tot = None
            for c in range(LANES):
                cvec = lanes_iota * 0 + c
                col = plsc.load_gather(stage, [lanes_iota, cvec])
                tot = col if tot is None else tot + col
            out_v[pl.ds(k0, LANES)] = tot
            return 0

        lax.fori_loop(0, n_groups, group_body, 0)

        for slot in range(NBUF):
            wait(slot)

        pltpu.sync_copy(out_v, out_hbm.at[pl.ds(base, bpw)])

    return kern(u_start, u_col, i_start, i_col, users_t, items_t)


def kernel(x, users_weight, items_weight):
    x32 = x.astype(jnp.int32)
    u = x32[:, 0]
    it = x32[:, 1]
    return _mf_forward(u & ~127, u & 127, it & ~127, it & 127,
                       users_weight.T, items_weight.T)
